# baseline (speedup 1.0000x reference)
import jax
import jax.numpy as jnp
from jax.experimental import pallas as pl

_HIGHEST = jax.lax.Precision.HIGHEST
_INTERPRET = False


def _mm(a, w):
    return jnp.matmul(a, w, precision=_HIGHEST)


def _scalar_tile(val):
    r = jax.lax.broadcasted_iota(jnp.int32, (8, 128), 0)
    c = jax.lax.broadcasted_iota(jnp.int32, (8, 128), 1)
    return jnp.where((r == 0) & (c == 0), val, 0.0)


def _k1(p_ref, w_ref, b_ref, o_ref):
    p = p_ref[0]
    acc = _mm(p, w_ref[...])
    o_ref[0] = jnp.maximum(acc + b_ref[...], 0.0)


def _k2(h_ref, w_ref, b_ref, o_ref):
    for r0 in range(0, 96, 24):
        acc = jnp.zeros((24, 96, 128), jnp.float32)
        for pi in range(2):
            for pj in range(2):
                hp = h_ref[0, pi, pj, r0:r0 + 25]
                f = _mm(hp.reshape(25 * 104, 64), w_ref[pi, pj])
                f = f.reshape(25, 104, 512)
                for di in range(2):
                    for dj in range(2):
                        c = (di * 2 + dj) * 128
                        acc = acc + f[di:di + 24, dj:dj + 96, c:c + 128]
        o_ref[0, r0:r0 + 24] = jnp.maximum(acc + b_ref[...], 0.0)


def _conv3x3_band(zp_ref, w_ref, b_ref, r0):
    acc = jnp.zeros((24, 96, 128), jnp.float32)
    for kh in range(3):
        rows = zp_ref[0, r0 + kh:r0 + kh + 24]
        f = _mm(rows.reshape(24 * 104, 128), w_ref[kh])
        f = f.reshape(24, 104, 384)
        for kw in range(3):
            acc = acc + f[:, kw:kw + 96, kw * 128:kw * 128 + 128]
    return acc + b_ref[...]


def _k3(zp_ref, w_ref, b_ref, cb_ref, bn_ref, zq_ref, vq_ref):
    bnorm = bn_ref[...]
    cb = cb_ref[...]
    vq_acc = jnp.zeros((), jnp.float32)
    for band in range(4):
        z = _conv3x3_band(zp_ref, w_ref, b_ref, band * 24)
        zc = z.reshape(2304, 128)
        a = jnp.sum(zc * zc, axis=1, keepdims=True)
        dist = a + bnorm
        m = jnp.min(dist, axis=1, keepdims=True)
        iota = jax.lax.broadcasted_iota(jnp.int32, (2304, 512), 1)
        idx = jnp.min(jnp.where(dist == m, iota, 512), axis=1)
        onehot = (iota == idx[:, None]).astype(jnp.float32)
        zq = _mm(onehot, cb)
        d = zq - zc
        vq_acc = vq_acc + jnp.sum(d * d)
        zq_ref[0, band * 2304:(band + 1) * 2304] = zq
    vq_ref[0] = _scalar_tile(vq_acc)


def _k4(zp_ref, w_ref, b_ref, o_ref):
    for band in range(4):
        o_ref[0, band * 24:(band + 1) * 24] = _conv3x3_band(
            zp_ref, w_ref, b_ref, band * 24)


def _k5(dp_ref, w_ref, b_ref, o_ref):
    for pi in range(2):
        for pj in range(2):
            for r0 in range(0, 96, 24):
                rows = dp_ref[0, r0 + pi:r0 + pi + 25]
                f = _mm(rows.reshape(25 * 104, 128), w_ref[pi, pj])
                f = f.reshape(25, 104, 256)
                acc = jnp.zeros((24, 96, 64), jnp.float32)
                for di in range(2):
                    for dj in range(2):
                        c = (di * 2 + dj) * 64
                        acc = acc + f[di:di + 24,
                                      dj + pj:dj + pj + 96, c:c + 64]
                o_ref[0, pi, pj, r0:r0 + 24] = jnp.maximum(
                    acc + b_ref[...], 0.0)


def _k6(hp_ref, x_ref, w_ref, b_ref, o_ref, rl_ref):
    bias = b_ref[0, 0]
    rl_acc = jnp.zeros((), jnp.float32)
    for qi in range(2):
        for qj in range(2):
            for r0 in range(0, 192, 48):
                acc = jnp.zeros((48, 192), jnp.float32)
                for e in range(2):
                    for f2 in range(2):
                        sl = hp_ref[0, e + qi + r0:e + qi + r0 + 48,
                                    f2 + qj:f2 + qj + 192]
                        wv = w_ref[2 * e + qi, 2 * f2 + qj]
                        acc = acc + jnp.sum(sl * wv[None, None, :], axis=-1)
                rec = acc + bias
                diff = rec - x_ref[0, qi, qj, r0:r0 + 48]
                rl_acc = rl_acc + jnp.sum(diff * diff)
                o_ref[0, qi, qj, r0:r0 + 48] = rec
    rl_ref[0] = _scalar_tile(rl_acc)


def _full(shape):
    n = len(shape)
    return pl.BlockSpec(shape, lambda i: (0,) * n)


def _batched(shape):
    n = len(shape)
    return pl.BlockSpec((1,) + shape, lambda i: (i,) + (0,) * n)


def _call(body, in_specs, out_specs, out_shapes, inputs):
    return pl.pallas_call(
        body,
        grid=(4,),
        in_specs=in_specs,
        out_specs=out_specs,
        out_shape=out_shapes,
        interpret=_INTERPRET,
    )(*inputs)


def kernel(x, enc_w1, enc_b1, enc_w2, enc_b2, enc_w3, enc_b3, codebook,
           dec_w1, dec_b1, dec_wt1, dec_bt1, dec_wt2, dec_bt2):
    f32 = jnp.float32

    w1r = enc_w1[:, 0].reshape(64, 16).T
    w1blk = jnp.kron(jnp.eye(8, dtype=f32), w1r)
    b1r = jnp.tile(enc_b1, 8)[None, :]
    w2r = jnp.stack([
        jnp.stack([
            jnp.concatenate([
                enc_w2[:, :, 2 * di + pi, 2 * dj + pj].T
                for di in range(2) for dj in range(2)], axis=1)
            for pj in range(2)], axis=0)
        for pi in range(2)], axis=0)
    w3r = jnp.stack([
        jnp.concatenate([enc_w3[:, :, kh, kw].T for kw in range(3)], axis=1)
        for kh in range(3)], axis=0)
    w4r = jnp.stack([
        jnp.concatenate([dec_w1[:, :, kh, kw].T for kw in range(3)], axis=1)
        for kh in range(3)], axis=0)
    wt1f = jnp.flip(dec_wt1, (2, 3))
    w5r = jnp.stack([
        jnp.stack([
            jnp.concatenate([
                wt1f[:, :, 2 * di + pi, 2 * dj + pj]
                for di in range(2) for dj in range(2)], axis=1)
            for pj in range(2)], axis=0)
        for pi in range(2)], axis=0)
    w6r = jnp.flip(dec_wt2, (2, 3))[:, 0].transpose(1, 2, 0)
    bnorm = jnp.sum(codebook ** 2, axis=1)[None, :]

    xp = jnp.pad(x[:, 0], ((0, 0), (1, 1), (1, 1)))
    xb = xp.reshape(4, 193, 2, 193, 2).transpose(0, 2, 4, 1, 3)
    p1 = jnp.stack([
        xb[:, kh % 2, kw % 2, kh // 2:kh // 2 + 192, kw // 2:kw // 2 + 192]
        for kh in range(4) for kw in range(4)], axis=-1)
    p1 = p1.reshape(4, 4608, 128)
    h1 = _call(_k1,
               [_batched((4608, 128)), _full((128, 512)), _full((1, 512))],
               _batched((4608, 512)),
               jax.ShapeDtypeStruct((4, 4608, 512), f32),
               (p1, w1blk, b1r))

    h1p = jnp.pad(h1.reshape(4, 192, 192, 64),
                  ((0, 0), (1, 1), (1, 1), (0, 0)))
    h1b = h1p.reshape(4, 97, 2, 97, 2, 64).transpose(0, 2, 4, 1, 3, 5)
    h1b = jnp.pad(h1b, ((0, 0), (0, 0), (0, 0), (0, 0), (0, 7), (0, 0)))
    h2 = _call(_k2,
               [_batched((2, 2, 97, 104, 64)), _full((2, 2, 64, 512)),
                _full((1, 128))],
               _batched((96, 96, 128)),
               jax.ShapeDtypeStruct((4, 96, 96, 128), f32),
               (h1b, w2r, enc_b2[None, :]))

    h2p = jnp.pad(h2, ((0, 0), (1, 1), (1, 7), (0, 0)))
    zq, vqp = _call(
        _k3,
        [_batched((98, 104, 128)), _full((3, 128, 384)), _full((1, 128)),
         _full((512, 128)), _full((1, 512))],
        [_batched((9216, 128)), _batched((8, 128))],
        (jax.ShapeDtypeStruct((4, 9216, 128), f32),
         jax.ShapeDtypeStruct((4, 8, 128), f32)),
        (h2p, w3r, enc_b3[None, :], codebook, bnorm))

    zqp = jnp.pad(zq.reshape(4, 96, 96, 128),
                  ((0, 0), (1, 1), (1, 7), (0, 0)))
    d1 = _call(_k4,
               [_batched((98, 104, 128)), _full((3, 128, 384)),
                _full((1, 128))],
               _batched((96, 96, 128)),
               jax.ShapeDtypeStruct((4, 96, 96, 128), f32),
               (zqp, w4r, dec_b1[None, :]))

    d1p = jnp.pad(d1, ((0, 0), (1, 1), (1, 7), (0, 0)))
    hph = _call(_k5,
                [_batched((98, 104, 128)), _full((2, 2, 128, 256)),
                 _full((1, 64))],
                _batched((2, 2, 96, 96, 64)),
                jax.ShapeDtypeStruct((4, 2, 2, 96, 96, 64), f32),
                (d1p, w5r, dec_bt1[None, :]))
    hmerge = hph.transpose(0, 3, 1, 4, 2, 5).reshape(4, 192, 192, 64)
    hpad = jnp.pad(hmerge, ((0, 0), (1, 1), (1, 1), (0, 0)))

    xs = x[:, 0].reshape(4, 192, 2, 192, 2).transpose(0, 2, 4, 1, 3)
    rec_ph, rlp = _call(
        _k6,
        [_batched((194, 194, 64)), _batched((2, 2, 192, 192)),
         _full((4, 4, 64)), _full((1, 1))],
        [_batched((2, 2, 192, 192)), _batched((8, 128))],
        (jax.ShapeDtypeStruct((4, 2, 2, 192, 192), f32),
         jax.ShapeDtypeStruct((4, 8, 128), f32)),
        (hpad, xs, w6r, dec_bt2[:, None]))

    recon = rec_ph.transpose(0, 3, 1, 4, 2).reshape(4, 1, 384, 384)
    reconstruction_loss = jnp.sum(rlp) / f32(589824.0)
    cb_loss = jnp.sum(vqp) / f32(4718592.0)
    vq_loss = cb_loss + cb_loss
    total_loss = reconstruction_loss + vq_loss
    return (recon, total_loss, reconstruction_loss, vq_loss)
